# Initial kernel scaffold; baseline (speedup 1.0000x reference)
import jax
import jax.numpy as jnp
from jax.experimental import pallas as pl


def kernel(boxs, numbers, coords, nuww0, sigmas0, centres0):
    raise NotImplementedError("write your pallas kernel here")


# speedup vs baseline: 101.5621x; 101.5621x over previous
import functools

import jax
import jax.numpy as jnp
from jax.experimental import pallas as pl

RCUT = 1.0
BI = 256
BJ = 128


def _desc_kernel(xi_ref, yi_ref, zi_ref, ni_ref,
                 xj_ref, yj_ref, zj_ref, nj_ref,
                 ww_ref, sg_ref, cen_ref, out_ref, *, n_atoms, n_feat, s_types):
    f32 = jnp.float32
    xi = xi_ref[0]
    yi = yi_ref[0]
    zi = zi_ref[0]
    ni = ni_ref[0]

    sg_all = sg_ref[...]
    ww_all = ww_ref[...]
    cs_all = cen_ref[...] * sg_all

    rcs, sgt, wwt = [], [], []
    for t in range(s_types):
        acc_c = jnp.zeros((xi.shape[0], n_feat), f32)
        acc_s = jnp.zeros((xi.shape[0], 1), f32)
        acc_w = jnp.zeros((xi.shape[0], 1), f32)
        for u in range(s_types):
            mu = (ni == u).astype(f32)
            z = s_types * u + t
            acc_c = acc_c + mu * cs_all[z, :][None, :]
            acc_s = acc_s + mu * sg_all[z, 0]
            acc_w = acc_w + mu * ww_all[z, 0]
        rcs.append(acc_c)
        sgt.append(acc_s)
        wwt.append(acc_w)

    i0 = pl.program_id(1) * xi.shape[0]
    irow = i0 + jax.lax.broadcasted_iota(jnp.int32, (xi.shape[0], 1), 0)

    def jtile(jt, acc):
        sl = pl.ds(jt * BJ, BJ)
        xj = xj_ref[0, :, sl]
        yj = yj_ref[0, :, sl]
        zj = zj_ref[0, :, sl]
        nj = nj_ref[0, :, sl]

        dx = xi - xj
        dy = yi - yj
        dz = zi - zj
        d2 = dx * dx + dy * dy + dz * dz
        d = jnp.sqrt(d2)

        jcol = jt * BJ + jax.lax.broadcasted_iota(jnp.int32, (1, BJ), 1)
        valid = (d2 <= RCUT * RCUT) & (irow != jcol)
        fc = 0.5 * jnp.cos(d * (jnp.pi / RCUT)) + 0.5

        mt = [(nj == t).astype(f32) for t in range(s_types)]
        sgmat = mt[0] * sgt[0]
        wwmat = mt[0] * wwt[0]
        for t in range(1, s_types):
            sgmat = sgmat + mt[t] * sgt[t]
            wwmat = wwmat + mt[t] * wwt[t]

        p = jnp.where(valid, wwmat * fc, 0.0)
        dsg = d * sgmat

        cols = []
        for c in range(n_feat):
            ccs = mt[0] * rcs[0][:, c:c + 1]
            for t in range(1, s_types):
                ccs = ccs + mt[t] * rcs[t][:, c:c + 1]
            a = dsg - ccs
            e = jnp.exp(-(a * a))
            cols.append(jnp.sum(p * e, axis=1, keepdims=True))
        return acc + jnp.concatenate(cols, axis=1)

    acc0 = jnp.zeros((xi.shape[0], n_feat), f32)
    out_ref[0] = jax.lax.fori_loop(0, n_atoms // BJ, jtile, acc0)


def kernel(boxs, numbers, coords, nuww0, sigmas0, centres0):
    b, n = numbers.shape
    s2 = nuww0.shape[0]
    s = int(round(s2 ** 0.5))
    c_feat = centres0.shape[1]

    xyz = coords.reshape(b, n, 3)
    x = xyz[:, :, 0]
    y = xyz[:, :, 1]
    z = xyz[:, :, 2]
    nb = numbers.astype(jnp.int32)

    col = lambda a: a[:, :, None]
    row = lambda a: a[:, None, :]

    grid = (b, n // BI)
    ispec = pl.BlockSpec((1, BI, 1), lambda bi, ii: (bi, ii, 0))
    jspec = pl.BlockSpec((1, 1, n), lambda bi, ii: (bi, 0, 0))
    tspec = pl.BlockSpec((s2, 1), lambda bi, ii: (0, 0))
    cspec = pl.BlockSpec((s2, c_feat), lambda bi, ii: (0, 0))
    ospec = pl.BlockSpec((1, BI, c_feat), lambda bi, ii: (bi, ii, 0))

    out = pl.pallas_call(
        functools.partial(_desc_kernel, n_atoms=n, n_feat=c_feat, s_types=s),
        grid=grid,
        in_specs=[ispec, ispec, ispec, ispec,
                  jspec, jspec, jspec, jspec,
                  tspec, tspec, cspec],
        out_specs=ospec,
        out_shape=jax.ShapeDtypeStruct((b, n, c_feat), jnp.float32),
    )(col(x), col(y), col(z), col(nb),
      row(x), row(y), row(z), row(nb),
      nuww0[:, None], sigmas0[:, None], centres0)
    return out


# speedup vs baseline: 208.8491x; 2.0564x over previous
import functools

import jax
import jax.numpy as jnp
from jax.experimental import pallas as pl
from jax.experimental.pallas import tpu as pltpu

RCUT = 1.0
BI = 256
BJ = 128
FAR = 1e6


def _desc_kernel(xi_ref, yi_ref, zi_ref, ni_ref,
                 xj_ref, yj_ref, zj_ref, jid_ref, tlab_ref,
                 ww_ref, sg_ref, cen_ref, out_ref,
                 *, n_tiles, n_feat, s_types):
    f32 = jnp.float32
    xi = xi_ref[0]
    yi = yi_ref[0]
    zi = zi_ref[0]
    ni = ni_ref[0]

    sg_all = sg_ref[...]
    ww_all = ww_ref[...]
    cs_all = cen_ref[...] * sg_all

    rcs, sgt, wwt = [], [], []
    for t in range(s_types):
        acc_c = jnp.zeros((xi.shape[0], n_feat), f32)
        acc_s = jnp.zeros((xi.shape[0], 1), f32)
        acc_w = jnp.zeros((xi.shape[0], 1), f32)
        for u in range(s_types):
            mu = (ni == u).astype(f32)
            z = s_types * u + t
            acc_c = acc_c + mu * cs_all[z, :][None, :]
            acc_s = acc_s + mu * sg_all[z, 0]
            acc_w = acc_w + mu * ww_all[z, 0]
        rcs.append(acc_c)
        sgt.append(acc_s)
        wwt.append(acc_w)

    i0 = pl.program_id(1) * xi.shape[0]
    irow = i0 + jax.lax.broadcasted_iota(jnp.int32, (xi.shape[0], 1), 0)
    ones = jnp.ones((BJ, 1), f32)

    def jtile(jt, acc):
        sl = pl.ds(jt * BJ, BJ)
        xj = xj_ref[0, :, sl]
        yj = yj_ref[0, :, sl]
        zj = zj_ref[0, :, sl]
        jid = jid_ref[0, :, sl]

        t = tlab_ref[0, 0, jt]
        st = [(t == u).astype(f32) for u in range(s_types)]
        ccs_mat = st[0] * rcs[0]
        sgcol = st[0] * sgt[0]
        wwcol = st[0] * wwt[0]
        for u in range(1, s_types):
            ccs_mat = ccs_mat + st[u] * rcs[u]
            sgcol = sgcol + st[u] * sgt[u]
            wwcol = wwcol + st[u] * wwt[u]

        dx = xi - xj
        dy = yi - yj
        dz = zi - zj
        d2 = dx * dx + dy * dy + dz * dz
        d = jnp.sqrt(d2)

        valid = (d2 <= RCUT * RCUT) & (irow != jid)
        fc = 0.5 * jnp.cos(d * (jnp.pi / RCUT)) + 0.5
        p = jnp.where(valid, wwcol * fc, 0.0)
        dsg = d * sgcol

        cols = []
        for c in range(n_feat):
            a = dsg - ccs_mat[:, c:c + 1]
            e = jnp.exp(-(a * a))
            pe = p * e
            cols.append(jax.lax.dot_general(
                pe, ones, (((1,), (0,)), ((), ())),
                preferred_element_type=f32))
        return acc + jnp.concatenate(cols, axis=1)

    acc0 = jnp.zeros((xi.shape[0], n_feat), f32)
    out_ref[0] = jax.lax.fori_loop(0, n_tiles, jtile, acc0)


def kernel(boxs, numbers, coords, nuww0, sigmas0, centres0):
    b, n = numbers.shape
    s2 = nuww0.shape[0]
    s = int(round(s2 ** 0.5))
    c_feat = centres0.shape[1]
    nt = n // BJ + s
    npad = nt * BJ

    xyz = coords.reshape(b, n, 3)
    x = xyz[:, :, 0]
    y = xyz[:, :, 1]
    z = xyz[:, :, 2]
    nb = numbers.astype(jnp.int32)

    order = jnp.argsort(nb, axis=1)
    bix = jnp.arange(b)[:, None]
    ns = jnp.take_along_axis(nb, order, axis=1)
    cnt = jnp.sum(nb[:, :, None] == jnp.arange(s)[None, None, :], axis=1)
    tiles_per = (cnt + BJ - 1) // BJ
    toff = jnp.concatenate(
        [jnp.zeros((b, 1), jnp.int32),
         jnp.cumsum(tiles_per[:, :-1], axis=1) * BJ], axis=1)
    cumcnt = jnp.concatenate(
        [jnp.zeros((b, 1), jnp.int32), jnp.cumsum(cnt[:, :-1], axis=1)], axis=1)
    rank = jnp.arange(n)[None, :] - jnp.take_along_axis(cumcnt, ns, axis=1)
    dst = jnp.take_along_axis(toff, ns, axis=1) + rank

    def scatter(vals, fill, dtype):
        out = jnp.full((b, npad), fill, dtype)
        return out.at[bix, dst].set(vals.astype(dtype))

    xp = scatter(jnp.take_along_axis(x, order, axis=1), FAR, jnp.float32)
    yp = scatter(jnp.take_along_axis(y, order, axis=1), FAR, jnp.float32)
    zp = scatter(jnp.take_along_axis(z, order, axis=1), FAR, jnp.float32)
    jid = scatter(order.astype(jnp.int32), -1, jnp.int32)

    tstart = jnp.arange(nt)[None, :] * BJ
    inseg = (tstart[:, :, None] >= toff[:, None, :]) & \
            (tstart[:, :, None] < (toff + tiles_per * BJ)[:, None, :])
    tlab = jnp.sum(inseg * jnp.arange(s)[None, None, :], axis=2).astype(jnp.int32)

    col = lambda a: a[:, :, None]
    row = lambda a: a[:, None, :]

    grid = (b, n // BI)
    ispec = pl.BlockSpec((1, BI, 1), lambda bi, ii: (bi, ii, 0))
    jspec = pl.BlockSpec((1, 1, npad), lambda bi, ii: (bi, 0, 0))
    lspec = pl.BlockSpec((1, 1, nt), lambda bi, ii: (bi, 0, 0),
                         memory_space=pltpu.SMEM)
    tspec = pl.BlockSpec((s2, 1), lambda bi, ii: (0, 0))
    cspec = pl.BlockSpec((s2, c_feat), lambda bi, ii: (0, 0))
    ospec = pl.BlockSpec((1, BI, c_feat), lambda bi, ii: (bi, ii, 0))

    out = pl.pallas_call(
        functools.partial(_desc_kernel, n_tiles=nt, n_feat=c_feat, s_types=s),
        grid=grid,
        in_specs=[ispec, ispec, ispec, ispec,
                  jspec, jspec, jspec, jspec, lspec,
                  tspec, tspec, cspec],
        out_specs=ospec,
        out_shape=jax.ShapeDtypeStruct((b, n, c_feat), jnp.float32),
    )(col(x), col(y), col(z), col(nb),
      row(xp), row(yp), row(zp), row(jid), tlab[:, None, :],
      nuww0[:, None], sigmas0[:, None], centres0)
    return out


# speedup vs baseline: 617.4534x; 2.9565x over previous
import functools

import jax
import jax.numpy as jnp
from jax.experimental import pallas as pl
from jax.experimental.pallas import tpu as pltpu

RCUT = 1.0
BI = 256
BJ = 128
FAR = 1e6
K = 16
NN = 32

_HI = jax.lax.Precision.HIGHEST


def _desc_kernel(xi_ref, yi_ref, zi_ref, ni_ref,
                 xj_ref, yj_ref, zj_ref, jid_ref, tlab_ref,
                 ww_ref, sg_ref, cen_ref, out_ref, a_ref,
                 *, n_tiles, n_feat, s_types):
    f32 = jnp.float32
    s2 = s_types * s_types
    bi = xi_ref.shape[1]

    first = (pl.program_id(0) == 0) & (pl.program_id(1) == 0)

    @pl.when(first)
    def _fit():
        rows = s2 * NN
        ridx = jax.lax.broadcasted_iota(jnp.int32, (rows, 1), 0)
        zrow = ridx // NN
        nrow = ridx - zrow * NN
        th = (nrow.astype(f32) + 0.5) * (jnp.pi / NN)
        dn = (jnp.cos(th) + 1.0) * (0.5 * RCUT)
        sgrow = jnp.zeros((rows, 1), f32)
        wwrow = jnp.zeros((rows, 1), f32)
        cenrow = jnp.zeros((rows, n_feat), f32)
        for z in range(s2):
            mz = (zrow == z).astype(f32)
            sgrow = sgrow + mz * sg_ref[z, 0]
            wwrow = wwrow + mz * ww_ref[z, 0]
            cenrow = cenrow + mz * cen_ref[z, :][None, :]
        arg = sgrow * (dn - cenrow)
        ev = wwrow * jnp.exp(-(arg * arg))
        ki = jax.lax.broadcasted_iota(jnp.int32, (K, NN), 0)
        karr = ki.astype(f32)
        narr = jax.lax.broadcasted_iota(jnp.int32, (K, NN), 1).astype(f32)
        w = (2.0 / NN) * jnp.cos(karr * (narr + 0.5) * (jnp.pi / NN))
        w = w * jnp.where(ki == 0, 0.5, 1.0)
        for z in range(s2):
            ez = ev[z * NN:(z + 1) * NN, :]
            a_ref[z * K:(z + 1) * K, :] = jax.lax.dot_general(
                w, ez, (((1,), (0,)), ((), ())),
                preferred_element_type=f32, precision=_HI)

    xi = xi_ref[0]
    yi = yi_ref[0]
    zi = zi_ref[0]
    ni = ni_ref[0]

    i0 = pl.program_id(1) * bi
    irow = i0 + jax.lax.broadcasted_iota(jnp.int32, (bi, 1), 0)

    def jtile(jt, m4):
        sl = pl.ds(jt * BJ, BJ)
        xj = xj_ref[0, :, sl]
        yj = yj_ref[0, :, sl]
        zj = zj_ref[0, :, sl]
        jid = jid_ref[0, :, sl]
        t = tlab_ref[0, 0, jt]

        dx = xi - xj
        dy = yi - yj
        dz = zi - zj
        d2 = dx * dx + dy * dy + dz * dz
        d = jnp.minimum(jnp.sqrt(d2), RCUT)
        valid = (d2 <= RCUT * RCUT) & (irow != jid)
        p = jnp.where(valid, 0.5 * jnp.cos(d * (jnp.pi / RCUT)) + 0.5, 0.0)

        x = 2.0 * (d / RCUT) - 1.0
        x2 = x + x
        um2 = p
        um1 = p * x
        cols = [jnp.sum(um2, axis=1, keepdims=True),
                jnp.sum(um1, axis=1, keepdims=True)]
        for _ in range(2, K):
            u = x2 * um1 - um2
            cols.append(jnp.sum(u, axis=1, keepdims=True))
            um2, um1 = um1, u
        mtile = jnp.concatenate(cols, axis=1)

        st = [(t == u).astype(f32) for u in range(s_types)]
        contrib = jnp.concatenate([st[u] * mtile for u in range(s_types)],
                                  axis=1)
        return m4 + contrib

    m0 = jnp.zeros((bi, s_types * K), f32)
    m4 = jax.lax.fori_loop(0, n_tiles, jtile, m0)

    g = jnp.zeros((bi, n_feat), f32)
    for u in range(s_types):
        mu = (ni == u).astype(f32)
        for t in range(s_types):
            az = a_ref[(s_types * u + t) * K:(s_types * u + t + 1) * K, :]
            mt = m4[:, t * K:(t + 1) * K]
            g = g + mu * jax.lax.dot_general(
                mt, az, (((1,), (0,)), ((), ())),
                preferred_element_type=f32, precision=_HI)
    out_ref[0] = g


def kernel(boxs, numbers, coords, nuww0, sigmas0, centres0):
    b, n = numbers.shape
    s2 = nuww0.shape[0]
    s = int(round(s2 ** 0.5))
    c_feat = centres0.shape[1]
    nt = n // BJ + s
    npad = nt * BJ

    xyz = coords.reshape(b, n, 3)
    x = xyz[:, :, 0]
    y = xyz[:, :, 1]
    z = xyz[:, :, 2]
    nb = numbers.astype(jnp.int32)

    order = jnp.argsort(nb, axis=1)
    bix = jnp.arange(b)[:, None]
    ns = jnp.take_along_axis(nb, order, axis=1)
    cnt = jnp.sum(nb[:, :, None] == jnp.arange(s)[None, None, :], axis=1)
    tiles_per = (cnt + BJ - 1) // BJ
    toff = jnp.concatenate(
        [jnp.zeros((b, 1), jnp.int32),
         jnp.cumsum(tiles_per[:, :-1], axis=1) * BJ], axis=1)
    cumcnt = jnp.concatenate(
        [jnp.zeros((b, 1), jnp.int32), jnp.cumsum(cnt[:, :-1], axis=1)], axis=1)
    rank = jnp.arange(n)[None, :] - jnp.take_along_axis(cumcnt, ns, axis=1)
    dst = jnp.take_along_axis(toff, ns, axis=1) + rank

    def scatter(vals, fill, dtype):
        out = jnp.full((b, npad), fill, dtype)
        return out.at[bix, dst].set(vals.astype(dtype))

    xp = scatter(jnp.take_along_axis(x, order, axis=1), FAR, jnp.float32)
    yp = scatter(jnp.take_along_axis(y, order, axis=1), FAR, jnp.float32)
    zp = scatter(jnp.take_along_axis(z, order, axis=1), FAR, jnp.float32)
    jid = scatter(order.astype(jnp.int32), -1, jnp.int32)

    tstart = jnp.arange(nt)[None, :] * BJ
    inseg = (tstart[:, :, None] >= toff[:, None, :]) & \
            (tstart[:, :, None] < (toff + tiles_per * BJ)[:, None, :])
    tlab = jnp.sum(inseg * jnp.arange(s)[None, None, :], axis=2).astype(jnp.int32)

    col = lambda a: a[:, :, None]
    row = lambda a: a[:, None, :]

    grid = (b, n // BI)
    ispec = pl.BlockSpec((1, BI, 1), lambda bi_, ii: (bi_, ii, 0))
    jspec = pl.BlockSpec((1, 1, npad), lambda bi_, ii: (bi_, 0, 0))
    lspec = pl.BlockSpec((1, 1, nt), lambda bi_, ii: (bi_, 0, 0),
                         memory_space=pltpu.SMEM)
    tspec = pl.BlockSpec((s2, 1), lambda bi_, ii: (0, 0))
    cspec = pl.BlockSpec((s2, c_feat), lambda bi_, ii: (0, 0))
    ospec = pl.BlockSpec((1, BI, c_feat), lambda bi_, ii: (bi_, ii, 0))

    out = pl.pallas_call(
        functools.partial(_desc_kernel, n_tiles=nt, n_feat=c_feat, s_types=s),
        grid=grid,
        in_specs=[ispec, ispec, ispec, ispec,
                  jspec, jspec, jspec, jspec, lspec,
                  tspec, tspec, cspec],
        out_specs=ospec,
        out_shape=jax.ShapeDtypeStruct((b, n, c_feat), jnp.float32),
        scratch_shapes=[pltpu.VMEM((s2 * K, c_feat), jnp.float32)],
    )(col(x), col(y), col(z), col(nb),
      row(xp), row(yp), row(zp), row(jid), tlab[:, None, :],
      nuww0[:, None], sigmas0[:, None], centres0)
    return out


# speedup vs baseline: 929.8413x; 1.5059x over previous
import functools

import jax
import jax.numpy as jnp
from jax.experimental import pallas as pl
from jax.experimental.pallas import tpu as pltpu

RCUT = 1.0
BI = 256
BJ = 128
FAR = 1e6
K = 16
NN = 32

_HI = jax.lax.Precision.HIGHEST


def _desc_kernel(xi_ref, yi_ref, zi_ref, ni_ref,
                 xj_ref, yj_ref, zj_ref, jid_ref, meta_ref,
                 ww_ref, sg_ref, cen_ref, out_ref, a_ref,
                 *, n_tiles, n_feat, s_types):
    f32 = jnp.float32
    s2 = s_types * s_types
    bi = xi_ref.shape[1]

    first = (pl.program_id(0) == 0) & (pl.program_id(1) == 0)

    @pl.when(first)
    def _fit():
        rows = s2 * NN
        ridx = jax.lax.broadcasted_iota(jnp.int32, (rows, 1), 0)
        zrow = ridx // NN
        nrow = ridx - zrow * NN
        th = (nrow.astype(f32) + 0.5) * (jnp.pi / NN)
        dn = (jnp.cos(th) + 1.0) * (0.5 * RCUT)
        sgrow = jnp.zeros((rows, 1), f32)
        wwrow = jnp.zeros((rows, 1), f32)
        cenrow = jnp.zeros((rows, n_feat), f32)
        for z in range(s2):
            mz = (zrow == z).astype(f32)
            sgrow = sgrow + mz * sg_ref[z, 0]
            wwrow = wwrow + mz * ww_ref[z, 0]
            cenrow = cenrow + mz * cen_ref[z, :][None, :]
        arg = sgrow * (dn - cenrow)
        fcn = 0.5 * jnp.cos(dn * (jnp.pi / RCUT)) + 0.5
        ev = wwrow * fcn * jnp.exp(-(arg * arg))
        ki = jax.lax.broadcasted_iota(jnp.int32, (K, NN), 0)
        karr = ki.astype(f32)
        narr = jax.lax.broadcasted_iota(jnp.int32, (K, NN), 1).astype(f32)
        w = (2.0 / NN) * jnp.cos(karr * (narr + 0.5) * (jnp.pi / NN))
        w = w * jnp.where(ki == 0, 0.5, 1.0)
        for z in range(s2):
            ez = ev[z * NN:(z + 1) * NN, :]
            a_ref[z * K:(z + 1) * K, :] = jax.lax.dot_general(
                w, ez, (((1,), (0,)), ((), ())),
                preferred_element_type=f32, precision=_HI)

    xi = xi_ref[0]
    yi = yi_ref[0]
    zi = zi_ref[0]
    ni = ni_ref[0]

    i0 = pl.program_id(1) * bi
    irow = i0 + jax.lax.broadcasted_iota(jnp.int32, (bi, 1), 0)

    def jtile(jt, mk):
        sl = pl.ds(jt * BJ, BJ)
        xj = xj_ref[0, :, sl]
        yj = yj_ref[0, :, sl]
        zj = zj_ref[0, :, sl]
        jid = jid_ref[0, :, sl]

        dx = xi - xj
        dy = yi - yj
        dz = zi - zj
        d2 = dx * dx + dy * dy + dz * dz
        d = jnp.minimum(jnp.sqrt(d2), RCUT)
        valid = (d2 <= RCUT * RCUT) & (irow != jid)
        p = valid.astype(f32)

        x = 2.0 * (d / RCUT) - 1.0
        x2 = x + x
        um2 = p
        um1 = p * x
        cols = [jnp.sum(um2, axis=1, keepdims=True),
                jnp.sum(um1, axis=1, keepdims=True)]
        for _ in range(2, K):
            u = x2 * um1 - um2
            cols.append(jnp.sum(u, axis=1, keepdims=True))
            um2, um1 = um1, u
        return mk + jnp.concatenate(cols, axis=1)

    mparts = []
    for t in range(s_types):
        ts = meta_ref[0, 0, t]
        tc = meta_ref[0, 0, s_types + t]
        mparts.append(jax.lax.fori_loop(
            ts, ts + tc, jtile, jnp.zeros((bi, K), f32)))

    g = jnp.zeros((bi, n_feat), f32)
    for u in range(s_types):
        mu = (ni == u).astype(f32)
        for t in range(s_types):
            az = a_ref[(s_types * u + t) * K:(s_types * u + t + 1) * K, :]
            g = g + mu * jax.lax.dot_general(
                mparts[t], az, (((1,), (0,)), ((), ())),
                preferred_element_type=f32, precision=_HI)
    out_ref[0] = g


def kernel(boxs, numbers, coords, nuww0, sigmas0, centres0):
    b, n = numbers.shape
    s2 = nuww0.shape[0]
    s = int(round(s2 ** 0.5))
    c_feat = centres0.shape[1]
    nt = n // BJ + s
    npad = nt * BJ

    xyz = coords.reshape(b, n, 3)
    x = xyz[:, :, 0]
    y = xyz[:, :, 1]
    z = xyz[:, :, 2]
    nb = numbers.astype(jnp.int32)

    order = jnp.argsort(nb, axis=1)
    bix = jnp.arange(b)[:, None]
    ns = jnp.take_along_axis(nb, order, axis=1)
    cnt = jnp.sum(nb[:, :, None] == jnp.arange(s)[None, None, :], axis=1)
    tiles_per = (cnt + BJ - 1) // BJ
    toff = jnp.concatenate(
        [jnp.zeros((b, 1), jnp.int32),
         jnp.cumsum(tiles_per[:, :-1], axis=1) * BJ], axis=1)
    cumcnt = jnp.concatenate(
        [jnp.zeros((b, 1), jnp.int32), jnp.cumsum(cnt[:, :-1], axis=1)], axis=1)
    rank = jnp.arange(n)[None, :] - jnp.take_along_axis(cumcnt, ns, axis=1)
    dst = jnp.take_along_axis(toff, ns, axis=1) + rank

    def scatter(vals, fill, dtype):
        out = jnp.full((b, npad), fill, dtype)
        return out.at[bix, dst].set(vals.astype(dtype))

    xp = scatter(jnp.take_along_axis(x, order, axis=1), FAR, jnp.float32)
    yp = scatter(jnp.take_along_axis(y, order, axis=1), FAR, jnp.float32)
    zp = scatter(jnp.take_along_axis(z, order, axis=1), FAR, jnp.float32)
    jid = scatter(order.astype(jnp.int32), -1, jnp.int32)

    meta = jnp.concatenate([toff // BJ, tiles_per], axis=1).astype(jnp.int32)

    col = lambda a: a[:, :, None]
    row = lambda a: a[:, None, :]

    grid = (b, n // BI)
    ispec = pl.BlockSpec((1, BI, 1), lambda bi_, ii: (bi_, ii, 0))
    jspec = pl.BlockSpec((1, 1, npad), lambda bi_, ii: (bi_, 0, 0))
    lspec = pl.BlockSpec((1, 1, 2 * s), lambda bi_, ii: (bi_, 0, 0),
                         memory_space=pltpu.SMEM)
    tspec = pl.BlockSpec((s2, 1), lambda bi_, ii: (0, 0))
    cspec = pl.BlockSpec((s2, c_feat), lambda bi_, ii: (0, 0))
    ospec = pl.BlockSpec((1, BI, c_feat), lambda bi_, ii: (bi_, ii, 0))

    out = pl.pallas_call(
        functools.partial(_desc_kernel, n_tiles=nt, n_feat=c_feat, s_types=s),
        grid=grid,
        in_specs=[ispec, ispec, ispec, ispec,
                  jspec, jspec, jspec, jspec, lspec,
                  tspec, tspec, cspec],
        out_specs=ospec,
        out_shape=jax.ShapeDtypeStruct((b, n, c_feat), jnp.float32),
        scratch_shapes=[pltpu.VMEM((s2 * K, c_feat), jnp.float32)],
    )(col(x), col(y), col(z), col(nb),
      row(xp), row(yp), row(zp), row(jid), meta[:, None, :],
      nuww0[:, None], sigmas0[:, None], centres0)
    return out


# speedup vs baseline: 983.9536x; 1.0582x over previous
import functools

import jax
import jax.numpy as jnp
from jax.experimental import pallas as pl
from jax.experimental.pallas import tpu as pltpu

RCUT = 1.0
BI = 256
BJ = 128
FAR = 1e6
K = 16
NN = 32

_HI = jax.lax.Precision.HIGHEST


def _desc_kernel(xi_ref, yi_ref, zi_ref, ni_ref, iid_ref,
                 xj_ref, yj_ref, zj_ref, jid_ref, meta_ref,
                 ww_ref, sg_ref, cen_ref, out_ref, a_ref,
                 *, n_tiles, n_feat, s_types):
    f32 = jnp.float32
    s2 = s_types * s_types
    bi = xi_ref.shape[1]

    first = (pl.program_id(0) == 0) & (pl.program_id(1) == 0)

    @pl.when(first)
    def _fit():
        rows = s2 * NN
        ridx = jax.lax.broadcasted_iota(jnp.int32, (rows, 1), 0)
        zrow = ridx // NN
        nrow = ridx - zrow * NN
        th = (nrow.astype(f32) + 0.5) * (jnp.pi / NN)
        dn = (jnp.cos(th) + 1.0) * (0.5 * RCUT)
        sgrow = jnp.zeros((rows, 1), f32)
        wwrow = jnp.zeros((rows, 1), f32)
        cenrow = jnp.zeros((rows, n_feat), f32)
        for z in range(s2):
            mz = (zrow == z).astype(f32)
            sgrow = sgrow + mz * sg_ref[z, 0]
            wwrow = wwrow + mz * ww_ref[z, 0]
            cenrow = cenrow + mz * cen_ref[z, :][None, :]
        arg = sgrow * (dn - cenrow)
        fcn = 0.5 * jnp.cos(dn * (jnp.pi / RCUT)) + 0.5
        ev = wwrow * fcn * jnp.exp(-(arg * arg))
        ki = jax.lax.broadcasted_iota(jnp.int32, (K, NN), 0)
        karr = ki.astype(f32)
        narr = jax.lax.broadcasted_iota(jnp.int32, (K, NN), 1).astype(f32)
        w = (2.0 / NN) * jnp.cos(karr * (narr + 0.5) * (jnp.pi / NN))
        w = w * jnp.where(ki == 0, 0.5, 1.0)
        for z in range(s2):
            ez = ev[z * NN:(z + 1) * NN, :]
            a_ref[z * K:(z + 1) * K, :] = jax.lax.dot_general(
                w, ez, (((1,), (0,)), ((), ())),
                preferred_element_type=f32, precision=_HI)

    xi = xi_ref[0]
    yi = yi_ref[0]
    zi = zi_ref[0]
    ni = ni_ref[0]
    irow = iid_ref[0]

    def jtile(jt, mk):
        sl = pl.ds(jt * BJ, BJ)
        xj = xj_ref[0, :, sl]
        yj = yj_ref[0, :, sl]
        zj = zj_ref[0, :, sl]
        jid = jid_ref[0, :, sl]

        dx = xi - xj
        dy = yi - yj
        dz = zi - zj
        d2 = dx * dx + dy * dy + dz * dz
        d = jnp.minimum(jnp.sqrt(d2), RCUT)
        valid = (d2 <= RCUT * RCUT) & (irow != jid)
        p = valid.astype(f32)

        x = 2.0 * (d / RCUT) - 1.0
        x2 = x + x
        um2 = p
        um1 = p * x
        cols = [jnp.sum(um2, axis=1, keepdims=True),
                jnp.sum(um1, axis=1, keepdims=True)]
        for _ in range(2, K):
            u = x2 * um1 - um2
            cols.append(jnp.sum(u, axis=1, keepdims=True))
            um2, um1 = um1, u
        return mk + jnp.concatenate(cols, axis=1)

    mparts = []
    for t in range(s_types):
        ts = meta_ref[0, 0, 0, t]
        te = meta_ref[0, 0, 0, s_types + t]
        mparts.append(jax.lax.fori_loop(
            ts, te, jtile, jnp.zeros((bi, K), f32)))

    g = jnp.zeros((bi, n_feat), f32)
    for u in range(s_types):
        mu = (ni == u).astype(f32)
        for t in range(s_types):
            az = a_ref[(s_types * u + t) * K:(s_types * u + t + 1) * K, :]
            g = g + mu * jax.lax.dot_general(
                mparts[t], az, (((1,), (0,)), ((), ())),
                preferred_element_type=f32, precision=_HI)
    out_ref[0] = g


def kernel(boxs, numbers, coords, nuww0, sigmas0, centres0):
    b, n = numbers.shape
    s2 = nuww0.shape[0]
    s = int(round(s2 ** 0.5))
    c_feat = centres0.shape[1]
    nt = n // BJ + s
    npad = nt * BJ

    xyz = coords.reshape(b, n, 3)
    x = xyz[:, :, 0]
    y = xyz[:, :, 1]
    z = xyz[:, :, 2]
    nb = numbers.astype(jnp.int32)
    bix = jnp.arange(b)[:, None]

    iord = jnp.argsort(x, axis=1)
    tk = lambda a, o: jnp.take_along_axis(a, o, axis=1)
    xi_s, yi_s, zi_s, ni_s = tk(x, iord), tk(y, iord), tk(z, iord), tk(nb, iord)

    nb_x = tk(nb, iord)
    jord = tk(iord, jnp.argsort(nb_x, axis=1, stable=True))
    ns = tk(nb, jord)
    xsj = tk(x, jord)
    cnt = jnp.sum(nb[:, :, None] == jnp.arange(s)[None, None, :], axis=1)
    tiles_per = (cnt + BJ - 1) // BJ
    toff = jnp.concatenate(
        [jnp.zeros((b, 1), jnp.int32),
         jnp.cumsum(tiles_per[:, :-1], axis=1) * BJ], axis=1)
    cumcnt = jnp.concatenate(
        [jnp.zeros((b, 1), jnp.int32), jnp.cumsum(cnt[:, :-1], axis=1)], axis=1)
    rank = jnp.arange(n)[None, :] - jnp.take_along_axis(cumcnt, ns, axis=1)
    dst = jnp.take_along_axis(toff, ns, axis=1) + rank

    def scatter(vals, fill, dtype):
        out = jnp.full((b, npad), fill, dtype)
        return out.at[bix, dst].set(vals.astype(dtype))

    xp = scatter(xsj, FAR, jnp.float32)
    yp = scatter(tk(y, jord), FAR, jnp.float32)
    zp = scatter(tk(z, jord), FAR, jnp.float32)
    jid = scatter(jord.astype(jnp.int32), -1, jnp.int32)

    nblk = n // BI
    xmin = xi_s[:, ::BI]
    xmax = xi_s[:, BI - 1::BI]
    lo_t, hi_t = [], []
    for t in range(s):
        mt = (ns == t)
        lo_cnt = jnp.sum(mt[:, None, :] & (xsj[:, None, :] < (xmin - RCUT)[:, :, None]),
                         axis=2)
        hi_cnt = jnp.sum(mt[:, None, :] & (xsj[:, None, :] <= (xmax + RCUT)[:, :, None]),
                         axis=2)
        lo_t.append((toff[:, t:t + 1] + lo_cnt) // BJ)
        hi_t.append((toff[:, t:t + 1] + hi_cnt + BJ - 1) // BJ)
    meta = jnp.stack(lo_t + hi_t, axis=2).astype(jnp.int32)

    col = lambda a: a[:, :, None]
    row = lambda a: a[:, None, :]

    grid = (b, nblk)
    ispec = pl.BlockSpec((1, BI, 1), lambda bi_, ii: (bi_, ii, 0))
    jspec = pl.BlockSpec((1, 1, npad), lambda bi_, ii: (bi_, 0, 0))
    lspec = pl.BlockSpec((1, 1, 1, 2 * s), lambda bi_, ii: (bi_, ii, 0, 0),
                         memory_space=pltpu.SMEM)
    tspec = pl.BlockSpec((s2, 1), lambda bi_, ii: (0, 0))
    cspec = pl.BlockSpec((s2, c_feat), lambda bi_, ii: (0, 0))
    ospec = pl.BlockSpec((1, BI, c_feat), lambda bi_, ii: (bi_, ii, 0))

    out = pl.pallas_call(
        functools.partial(_desc_kernel, n_tiles=nt, n_feat=c_feat, s_types=s),
        grid=grid,
        in_specs=[ispec, ispec, ispec, ispec, ispec,
                  jspec, jspec, jspec, jspec, lspec,
                  tspec, tspec, cspec],
        out_specs=ospec,
        out_shape=jax.ShapeDtypeStruct((b, n, c_feat), jnp.float32),
        scratch_shapes=[pltpu.VMEM((s2 * K, c_feat), jnp.float32)],
    )(col(xi_s), col(yi_s), col(zi_s), col(ni_s), col(iord.astype(jnp.int32)),
      row(xp), row(yp), row(zp), row(jid), meta[:, :, None, :],
      nuww0[:, None], sigmas0[:, None], centres0)

    inv = jnp.argsort(iord, axis=1)
    return jnp.take_along_axis(out, inv[:, :, None], axis=1)
